# baseline (speedup 1.0000x reference)
import functools

import jax
import jax.numpy as jnp
from jax.experimental import pallas as pl
from jax.experimental.pallas import tpu as pltpu

P = 1024
K = 20
NB = 40
N = NB * P
NC = N // 2
PC = P // 2
E2 = NC * K

_INTERPRET = False


def _knn_msg_body(pos_ref, post_ref, w1_ref, b1_ref, w2_ref, g_ref,
                  fn_ref, *, pts):
    xc = pos_ref[0]
    xr = post_ref[0]
    sqc = jnp.zeros((pts, 1), jnp.float32)
    sqr = jnp.zeros((1, pts), jnp.float32)
    for d in range(8):
        col = xc[:, d:d + 1]
        row = xr[d:d + 1, :]
        sqc = sqc + col * col
        sqr = sqr + row * row
    dot = jnp.dot(xc.astype(jnp.bfloat16), xr.astype(jnp.bfloat16),
                  preferred_element_type=jnp.float32)
    d2 = (sqc + sqr) - 2.0 * dot
    ii = jax.lax.broadcasted_iota(jnp.int32, (pts, pts), 0)
    jj = jax.lax.broadcasted_iota(jnp.int32, (pts, pts), 1)
    d2 = d2 + jnp.where(ii == jj, jnp.float32(1e9), jnp.float32(0.0))
    jjf = jj.astype(jnp.float32)
    hi = xc.astype(jnp.bfloat16)
    hif = hi.astype(jnp.float32)
    lo = (xc - hif).astype(jnp.bfloat16)
    lof = lo.astype(jnp.float32)
    lo2 = (xc - hif - lof).astype(jnp.bfloat16)
    phml = jnp.concatenate([hi, lo, lo2], axis=1)
    w1b = w1_ref[...].astype(jnp.bfloat16)
    b1v = b1_ref[0]
    big = jnp.float32(3e9)
    s_sum = jnp.zeros((pts, 64), jnp.float32)
    s_sq = jnp.zeros((pts, 64), jnp.float32)
    s_max = jnp.full((pts, 64), -jnp.inf, jnp.float32)
    s_min = jnp.full((pts, 64), jnp.inf, jnp.float32)
    for t in range(K):
        m = jnp.min(d2, axis=1, keepdims=True)
        cand = jnp.where(d2 <= m, jjf, jnp.float32(pts))
        sel = jnp.min(cand, axis=1, keepdims=True)
        selmask = jjf == sel
        maskb = jnp.where(selmask, 1.0, 0.0).astype(jnp.bfloat16)
        d2 = jnp.where(selmask, big, d2)
        pg = jnp.dot(maskb, phml, preferred_element_type=jnp.float32)
        psrc = (pg[:, 0:8] + pg[:, 8:16]) + pg[:, 16:24]
        rel = psrc - xc
        msg = jnp.maximum(
            jnp.dot(rel.astype(jnp.bfloat16), w1b,
                    preferred_element_type=jnp.float32) + b1v, 0.0)
        s_sum += msg
        s_sq += msg * msg
        s_max = jnp.maximum(s_max, msg)
        s_min = jnp.minimum(s_min, msg)
    chunks = (s_max, s_min, s_sum, s_sum * (1.0 / K), s_sq * (1.0 / K),
              s_max - s_min)
    fg_chunks = []
    cols = []
    for ch in chunks:
        c3 = ch.reshape(pts // 2, 2, 64)
        fgc = jnp.maximum(c3[:, 0, :], c3[:, 1, :])
        fg_chunks.append(fgc)
    fg = jnp.concatenate(fg_chunks, axis=1)
    g_ref[0] = jnp.dot(fg, w2_ref[...],
                       preferred_element_type=jnp.float32,
                       precision=jax.lax.Precision.HIGHEST)
    for fgc in fg_chunks:
        cols.append(jnp.sqrt(jnp.sum(fgc * fgc, axis=1, keepdims=True)
                             + 1e-12))
    cols.append(jnp.zeros((pts // 2, 2), jnp.float32))
    fn_ref[0] = jnp.concatenate(cols, axis=1)


def _knn_msg(pos8, pos8t, w1, b1, w2):
    return pl.pallas_call(
        functools.partial(_knn_msg_body, pts=P),
        grid=(NB,),
        in_specs=[
            pl.BlockSpec((1, P, 8), lambda i: (i, 0, 0)),
            pl.BlockSpec((1, 8, P), lambda i: (i, 0, 0)),
            pl.BlockSpec((8, 64), lambda i: (0, 0)),
            pl.BlockSpec((1, 64), lambda i: (0, 0)),
            pl.BlockSpec((384, 128), lambda i: (0, 0)),
        ],
        out_specs=[
            pl.BlockSpec((1, PC, 128), lambda i: (i, 0, 0)),
            pl.BlockSpec((1, PC, 8), lambda i: (i, 0, 0)),
        ],
        out_shape=[
            jax.ShapeDtypeStruct((NB, PC, 128), jnp.float32),
            jax.ShapeDtypeStruct((NB, PC, 8), jnp.float32),
        ],
        compiler_params=pltpu.CompilerParams(
            dimension_semantics=("parallel",)),
        interpret=_INTERPRET,
    )(pos8, pos8t, w1, b1, w2)


def _knn_h_body(fn_ref, fnt_ref, g_ref, b2_ref, mx_ref, mn_ref, ps_ref):
    pts = PC
    xc = fn_ref[0]
    xr = fnt_ref[0]
    sqc = jnp.zeros((pts, 1), jnp.float32)
    sqr = jnp.zeros((1, pts), jnp.float32)
    for d in range(8):
        col = xc[:, d:d + 1]
        row = xr[d:d + 1, :]
        sqc = sqc + col * col
        sqr = sqr + row * row
    dot = jnp.dot(xc.astype(jnp.bfloat16), xr.astype(jnp.bfloat16),
                  preferred_element_type=jnp.float32)
    d2 = (sqc + sqr) - 2.0 * dot
    ii = jax.lax.broadcasted_iota(jnp.int32, (pts, pts), 0)
    jj = jax.lax.broadcasted_iota(jnp.int32, (pts, pts), 1)
    d2 = d2 + jnp.where(ii == jj, jnp.float32(1e9), jnp.float32(0.0))
    jjf = jj.astype(jnp.float32)
    gb = g_ref[...]
    ghi = gb.astype(jnp.bfloat16)
    glo = (gb - ghi.astype(jnp.float32)).astype(jnp.bfloat16)
    ghl = jnp.concatenate([ghi, glo], axis=1)
    b2v = b2_ref[0]
    big = jnp.float32(3e9)
    h_max = jnp.full((pts, 128), -jnp.inf, jnp.float32)
    h_min = jnp.full((pts, 128), jnp.inf, jnp.float32)
    h_sum = jnp.zeros((1, 128), jnp.float32)
    h_sq = jnp.zeros((1, 128), jnp.float32)
    for t in range(K):
        m = jnp.min(d2, axis=1, keepdims=True)
        cand = jnp.where(d2 <= m, jjf, jnp.float32(pts))
        sel = jnp.min(cand, axis=1, keepdims=True)
        selmask = jjf == sel
        maskb = jnp.where(selmask, 1.0, 0.0).astype(jnp.bfloat16)
        d2 = jnp.where(selmask, big, d2)
        pg = jnp.dot(maskb, ghl, preferred_element_type=jnp.float32)
        gsrc = pg[:, 0:128] + pg[:, 128:256]
        h = jnp.maximum(gsrc - gb + b2v, 0.0)
        h_max = jnp.maximum(h_max, h)
        h_min = jnp.minimum(h_min, h)
        h_sum += jnp.sum(h, axis=0, keepdims=True)
        h_sq += jnp.sum(h * h, axis=0, keepdims=True)
    mx_ref[...] = h_max
    mn_ref[...] = h_min
    ps_ref[...] = jnp.concatenate([h_sum, h_sq], axis=0)[None]


def _knn_h(fn8, fn8t, g, b2):
    return pl.pallas_call(
        _knn_h_body,
        grid=(NB,),
        in_specs=[
            pl.BlockSpec((1, PC, 8), lambda i: (i, 0, 0)),
            pl.BlockSpec((1, 8, PC), lambda i: (i, 0, 0)),
            pl.BlockSpec((PC, 128), lambda i: (i, 0)),
            pl.BlockSpec((1, 128), lambda i: (0, 0)),
        ],
        out_specs=[
            pl.BlockSpec((PC, 128), lambda i: (i, 0)),
            pl.BlockSpec((PC, 128), lambda i: (i, 0)),
            pl.BlockSpec((1, 2, 128), lambda i: (i, 0, 0)),
        ],
        out_shape=[
            jax.ShapeDtypeStruct((NC, 128), jnp.float32),
            jax.ShapeDtypeStruct((NC, 128), jnp.float32),
            jax.ShapeDtypeStruct((NB, 2, 128), jnp.float32),
        ],
        compiler_params=pltpu.CompilerParams(
            dimension_semantics=("parallel",)),
        interpret=_INTERPRET,
    )(fn8, fn8t, g, b2)


def _head1_body(mx_ref, mn_ref, ps_ref, gdd_ref, bdd_ref, wn1_ref, bl1_ref,
                y1p_ref):
    psum = jnp.sum(ps_ref[:, 0, :], axis=0, keepdims=True)
    psq = jnp.sum(ps_ref[:, 1, :], axis=0, keepdims=True)
    mean = psum * (1.0 / E2)
    var = psq * (1.0 / E2) - mean * mean
    s = gdd_ref[0:1, :] * jax.lax.rsqrt(var + 1e-5)
    tt = bdd_ref[0:1, :] - s * mean
    fd2 = jnp.where(s >= 0.0, s * mx_ref[...], s * mn_ref[...]) + tt
    y1 = jnp.dot(fd2, wn1_ref[...], preferred_element_type=jnp.float32)
    y1 = y1 + bl1_ref[0:1, :]
    y1p_ref[...] = jnp.max(y1, axis=0, keepdims=True)[None]


def _head1(mx, mn, ps, gdd, bdd, wn1, bl1):
    nb_ps = ps.shape[0]
    return pl.pallas_call(
        _head1_body,
        grid=(NB,),
        in_specs=[
            pl.BlockSpec((PC, 128), lambda i: (i, 0)),
            pl.BlockSpec((PC, 128), lambda i: (i, 0)),
            pl.BlockSpec((nb_ps, 2, 128), lambda i: (0, 0, 0)),
            pl.BlockSpec((1, 128), lambda i: (0, 0)),
            pl.BlockSpec((1, 128), lambda i: (0, 0)),
            pl.BlockSpec((128, 1024), lambda i: (0, 0)),
            pl.BlockSpec((1, 1024), lambda i: (0, 0)),
        ],
        out_specs=pl.BlockSpec((1, 1, 1024), lambda i: (i, 0, 0)),
        out_shape=jax.ShapeDtypeStruct((NB, 1, 1024), jnp.float32),
        compiler_params=pltpu.CompilerParams(
            dimension_semantics=("parallel",)),
        interpret=_INTERPRET,
    )(mx, mn, ps, gdd, bdd, wn1, bl1)


def _bn_rows(x, g, b):
    m = jnp.mean(x, axis=0, keepdims=True)
    v = jnp.mean(x * x, axis=0, keepdims=True) - m * m
    return g * (x - m) * jax.lax.rsqrt(v + 1e-5) + b


def _head2_body(y1p_ref, g1_ref, be1_ref, wn2_ref, bl2_ref, g2_ref, be2_ref,
                wn3_ref, bl3_ref, g3_ref, be3_ref, wn4_ref, bl4_ref, out_ref):
    z = jnp.maximum(y1p_ref[...], 0.0)
    z = _bn_rows(z, g1_ref[0:1, :], be1_ref[0:1, :])
    z = jnp.maximum(jnp.dot(z, wn2_ref[...],
                            preferred_element_type=jnp.float32)
                    + bl2_ref[0:1, :], 0.0)
    z = _bn_rows(z, g2_ref[0:1, :], be2_ref[0:1, :])
    z = jnp.maximum(jnp.dot(z, wn3_ref[...],
                            preferred_element_type=jnp.float32)
                    + bl3_ref[0:1, :], 0.0)
    z = _bn_rows(z, g3_ref[0:1, :], be3_ref[0:1, :])
    z = jnp.dot(z, wn4_ref[...], preferred_element_type=jnp.float32) \
        + bl4_ref[0:1, :]
    zmax = jnp.max(z, axis=1, keepdims=True)
    zs = z - zmax
    lse = jnp.log(jnp.sum(jnp.exp(zs), axis=1, keepdims=True))
    out_ref[...] = zs - lse


def _head2(y1p, g1, be1, wn2, bl2, g2, be2, wn3, bl3, g3, be3, wn4, bl4):
    full = lambda s: pl.BlockSpec(s, lambda: tuple(0 for _ in s))
    args = (y1p, g1.reshape(1, -1), be1.reshape(1, -1), wn2,
            bl2.reshape(1, -1), g2.reshape(1, -1), be2.reshape(1, -1), wn3,
            bl3.reshape(1, -1), g3.reshape(1, -1), be3.reshape(1, -1), wn4,
            bl4.reshape(1, -1))
    return pl.pallas_call(
        _head2_body,
        in_specs=[full(a.shape) for a in args],
        out_specs=full((NB, 40)),
        out_shape=jax.ShapeDtypeStruct((NB, 40), jnp.float32),
        interpret=_INTERPRET,
    )(*args)


def kernel(pos, edge_index, batch, W1, b1, W2, b2, gdd, bdd, Wn1, bl1, g1,
           be1, Wn2, bl2, g2, be2, Wn3, bl3, g3, be3, Wn4, bl4):
    del edge_index, batch
    posf = jnp.pad(pos, ((0, 0), (0, 5)))
    pos8 = posf.reshape(NB, P, 8)
    pos8t = pos8.transpose(0, 2, 1)
    w1p = jnp.pad(W1, ((0, 5), (0, 0)))
    g, fn8 = _knn_msg(pos8, pos8t, w1p, b1.reshape(1, 64), W2)
    g = g.reshape(NC, 128)
    mx, mn, ps = _knn_h(fn8, fn8.transpose(0, 2, 1), g, b2.reshape(1, 128))
    y1p = _head1(mx, mn, ps, gdd.reshape(1, 128), bdd.reshape(1, 128), Wn1,
                 bl1.reshape(1, 1024)).reshape(NB, 1024)
    return _head2(y1p, g1, be1, Wn2, bl2, g2, be2, Wn3, bl3, g3, be3, Wn4,
                  bl4)


# speedup vs baseline: 1.1238x; 1.1238x over previous
import functools

import jax
import jax.numpy as jnp
from jax.experimental import pallas as pl
from jax.experimental.pallas import tpu as pltpu

P = 1024
K = 20
NB = 40
N = NB * P
NC = N // 2
PC = P // 2
E2 = NC * K

_INTERPRET = False


def _knn_msg_body(pos_ref, post_ref, w1_ref, b1_ref, feat_ref, *, pts):
    xc = pos_ref[0]
    xr = post_ref[0]
    sqc = jnp.zeros((pts, 1), jnp.float32)
    sqr = jnp.zeros((1, pts), jnp.float32)
    for d in range(8):
        col = xc[:, d:d + 1]
        row = xr[d:d + 1, :]
        sqc = sqc + col * col
        sqr = sqr + row * row
    dot = jnp.dot(xc.astype(jnp.bfloat16), xr.astype(jnp.bfloat16),
                  preferred_element_type=jnp.float32)
    d2 = (sqc + sqr) - 2.0 * dot
    ii = jax.lax.broadcasted_iota(jnp.int32, (pts, pts), 0)
    jj = jax.lax.broadcasted_iota(jnp.int32, (pts, pts), 1)
    d2 = d2 + jnp.where(ii == jj, jnp.float32(1e9), jnp.float32(0.0))
    jjf = jj.astype(jnp.float32)
    hi = xc.astype(jnp.bfloat16)
    hif = hi.astype(jnp.float32)
    lo = (xc - hif).astype(jnp.bfloat16)
    lof = lo.astype(jnp.float32)
    lo2 = (xc - hif - lof).astype(jnp.bfloat16)
    phml = jnp.concatenate([hi, lo, lo2], axis=1)
    w1b = w1_ref[...].astype(jnp.bfloat16)
    b1v = b1_ref[0]
    big = jnp.float32(3e9)
    s_sum = jnp.zeros((pts, 64), jnp.float32)
    s_sq = jnp.zeros((pts, 64), jnp.float32)
    s_max = jnp.full((pts, 64), -jnp.inf, jnp.float32)
    s_min = jnp.full((pts, 64), jnp.inf, jnp.float32)
    for t in range(K):
        m = jnp.min(d2, axis=1, keepdims=True)
        cand = jnp.where(d2 <= m, jjf, jnp.float32(pts))
        sel = jnp.min(cand, axis=1, keepdims=True)
        selmask = jjf == sel
        maskb = jnp.where(selmask, 1.0, 0.0).astype(jnp.bfloat16)
        d2 = jnp.where(selmask, big, d2)
        pg = jnp.dot(maskb, phml, preferred_element_type=jnp.float32)
        psrc = (pg[:, 0:8] + pg[:, 8:16]) + pg[:, 16:24]
        rel = psrc - xc
        msg = jnp.maximum(
            jnp.dot(rel.astype(jnp.bfloat16), w1b,
                    preferred_element_type=jnp.float32) + b1v, 0.0)
        s_sum += msg
        s_sq += msg * msg
        s_max = jnp.maximum(s_max, msg)
        s_min = jnp.minimum(s_min, msg)
    feat_ref[0] = jnp.concatenate(
        [s_max, s_min, s_sum, s_sum * (1.0 / K), s_sq * (1.0 / K),
         s_max - s_min], axis=1)


def _knn_msg(pos8, pos8t, w1, b1):
    return pl.pallas_call(
        functools.partial(_knn_msg_body, pts=P),
        grid=(NB,),
        in_specs=[
            pl.BlockSpec((1, P, 8), lambda i: (i, 0, 0)),
            pl.BlockSpec((1, 8, P), lambda i: (i, 0, 0)),
            pl.BlockSpec((8, 64), lambda i: (0, 0)),
            pl.BlockSpec((1, 64), lambda i: (0, 0)),
        ],
        out_specs=pl.BlockSpec((1, P, 384), lambda i: (i, 0, 0)),
        out_shape=jax.ShapeDtypeStruct((NB, P, 384), jnp.float32),
        compiler_params=pltpu.CompilerParams(
            dimension_semantics=("parallel",)),
        interpret=_INTERPRET,
    )(pos8, pos8t, w1, b1)


BLKP = 512


def _pairmax_body(fp_ref, w2_ref, g_ref, fn_ref):
    fg = jnp.maximum(fp_ref[:, 0, :], fp_ref[:, 1, :])
    g_ref[...] = jnp.dot(fg, w2_ref[...],
                         preferred_element_type=jnp.float32,
                         precision=jax.lax.Precision.HIGHEST)
    cols = []
    for c in range(6):
        ch = fg[:, c * 64:(c + 1) * 64]
        cols.append(jnp.sqrt(jnp.sum(ch * ch, axis=1, keepdims=True)
                             + 1e-12))
    cols.append(jnp.zeros((BLKP, 2), jnp.float32))
    fn_ref[...] = jnp.concatenate(cols, axis=1)


def _pairmax(feat_pair, w2):
    return pl.pallas_call(
        _pairmax_body,
        grid=(NC // BLKP,),
        in_specs=[
            pl.BlockSpec((BLKP, 2, 384), lambda i: (i, 0, 0)),
            pl.BlockSpec((384, 128), lambda i: (0, 0)),
        ],
        out_specs=[
            pl.BlockSpec((BLKP, 128), lambda i: (i, 0)),
            pl.BlockSpec((BLKP, 8), lambda i: (i, 0)),
        ],
        out_shape=[
            jax.ShapeDtypeStruct((NC, 128), jnp.float32),
            jax.ShapeDtypeStruct((NC, 8), jnp.float32),
        ],
        compiler_params=pltpu.CompilerParams(
            dimension_semantics=("parallel",)),
        interpret=_INTERPRET,
    )(feat_pair, w2)


def _knn_h_body(fn_ref, fnt_ref, g_ref, b2_ref, mx_ref, mn_ref, ps_ref):
    pts = PC
    xc = fn_ref[0]
    xr = fnt_ref[0]
    sqc = jnp.zeros((pts, 1), jnp.float32)
    sqr = jnp.zeros((1, pts), jnp.float32)
    for d in range(8):
        col = xc[:, d:d + 1]
        row = xr[d:d + 1, :]
        sqc = sqc + col * col
        sqr = sqr + row * row
    dot = jnp.dot(xc.astype(jnp.bfloat16), xr.astype(jnp.bfloat16),
                  preferred_element_type=jnp.float32)
    d2 = (sqc + sqr) - 2.0 * dot
    ii = jax.lax.broadcasted_iota(jnp.int32, (pts, pts), 0)
    jj = jax.lax.broadcasted_iota(jnp.int32, (pts, pts), 1)
    d2 = d2 + jnp.where(ii == jj, jnp.float32(1e9), jnp.float32(0.0))
    jjf = jj.astype(jnp.float32)
    gb = g_ref[...]
    ghi = gb.astype(jnp.bfloat16)
    glo = (gb - ghi.astype(jnp.float32)).astype(jnp.bfloat16)
    ghl = jnp.concatenate([ghi, glo], axis=1)
    b2v = b2_ref[0]
    big = jnp.float32(3e9)
    h_max = jnp.full((pts, 128), -jnp.inf, jnp.float32)
    h_min = jnp.full((pts, 128), jnp.inf, jnp.float32)
    h_sum = jnp.zeros((1, 128), jnp.float32)
    h_sq = jnp.zeros((1, 128), jnp.float32)
    for t in range(K):
        m = jnp.min(d2, axis=1, keepdims=True)
        cand = jnp.where(d2 <= m, jjf, jnp.float32(pts))
        sel = jnp.min(cand, axis=1, keepdims=True)
        selmask = jjf == sel
        maskb = jnp.where(selmask, 1.0, 0.0).astype(jnp.bfloat16)
        d2 = jnp.where(selmask, big, d2)
        pg = jnp.dot(maskb, ghl, preferred_element_type=jnp.float32)
        gsrc = pg[:, 0:128] + pg[:, 128:256]
        h = jnp.maximum(gsrc - gb + b2v, 0.0)
        h_max = jnp.maximum(h_max, h)
        h_min = jnp.minimum(h_min, h)
        h_sum += jnp.sum(h, axis=0, keepdims=True)
        h_sq += jnp.sum(h * h, axis=0, keepdims=True)
    mx_ref[...] = h_max
    mn_ref[...] = h_min
    ps_ref[...] = jnp.concatenate([h_sum, h_sq], axis=0)[None]


def _knn_h(fn8, fn8t, g, b2):
    return pl.pallas_call(
        _knn_h_body,
        grid=(NB,),
        in_specs=[
            pl.BlockSpec((1, PC, 8), lambda i: (i, 0, 0)),
            pl.BlockSpec((1, 8, PC), lambda i: (i, 0, 0)),
            pl.BlockSpec((PC, 128), lambda i: (i, 0)),
            pl.BlockSpec((1, 128), lambda i: (0, 0)),
        ],
        out_specs=[
            pl.BlockSpec((PC, 128), lambda i: (i, 0)),
            pl.BlockSpec((PC, 128), lambda i: (i, 0)),
            pl.BlockSpec((1, 2, 128), lambda i: (i, 0, 0)),
        ],
        out_shape=[
            jax.ShapeDtypeStruct((NC, 128), jnp.float32),
            jax.ShapeDtypeStruct((NC, 128), jnp.float32),
            jax.ShapeDtypeStruct((NB, 2, 128), jnp.float32),
        ],
        compiler_params=pltpu.CompilerParams(
            dimension_semantics=("parallel",)),
        interpret=_INTERPRET,
    )(fn8, fn8t, g, b2)


def _head1_body(mx_ref, mn_ref, ps_ref, gdd_ref, bdd_ref, wn1_ref, bl1_ref,
                y1p_ref):
    psum = jnp.sum(ps_ref[:, 0, :], axis=0, keepdims=True)
    psq = jnp.sum(ps_ref[:, 1, :], axis=0, keepdims=True)
    mean = psum * (1.0 / E2)
    var = psq * (1.0 / E2) - mean * mean
    s = gdd_ref[0:1, :] * jax.lax.rsqrt(var + 1e-5)
    tt = bdd_ref[0:1, :] - s * mean
    fd2 = jnp.where(s >= 0.0, s * mx_ref[...], s * mn_ref[...]) + tt
    y1 = jnp.dot(fd2, wn1_ref[...], preferred_element_type=jnp.float32)
    y1 = y1 + bl1_ref[0:1, :]
    y1p_ref[...] = jnp.max(y1, axis=0, keepdims=True)[None]


def _head1(mx, mn, ps, gdd, bdd, wn1, bl1):
    nb_ps = ps.shape[0]
    return pl.pallas_call(
        _head1_body,
        grid=(NB,),
        in_specs=[
            pl.BlockSpec((PC, 128), lambda i: (i, 0)),
            pl.BlockSpec((PC, 128), lambda i: (i, 0)),
            pl.BlockSpec((nb_ps, 2, 128), lambda i: (0, 0, 0)),
            pl.BlockSpec((1, 128), lambda i: (0, 0)),
            pl.BlockSpec((1, 128), lambda i: (0, 0)),
            pl.BlockSpec((128, 1024), lambda i: (0, 0)),
            pl.BlockSpec((1, 1024), lambda i: (0, 0)),
        ],
        out_specs=pl.BlockSpec((1, 1, 1024), lambda i: (i, 0, 0)),
        out_shape=jax.ShapeDtypeStruct((NB, 1, 1024), jnp.float32),
        compiler_params=pltpu.CompilerParams(
            dimension_semantics=("parallel",)),
        interpret=_INTERPRET,
    )(mx, mn, ps, gdd, bdd, wn1, bl1)


def _bn_rows(x, g, b):
    m = jnp.mean(x, axis=0, keepdims=True)
    v = jnp.mean(x * x, axis=0, keepdims=True) - m * m
    return g * (x - m) * jax.lax.rsqrt(v + 1e-5) + b


def _head2_body(y1p_ref, g1_ref, be1_ref, wn2_ref, bl2_ref, g2_ref, be2_ref,
                wn3_ref, bl3_ref, g3_ref, be3_ref, wn4_ref, bl4_ref, out_ref):
    z = jnp.maximum(y1p_ref[...], 0.0)
    z = _bn_rows(z, g1_ref[0:1, :], be1_ref[0:1, :])
    z = jnp.maximum(jnp.dot(z, wn2_ref[...],
                            preferred_element_type=jnp.float32)
                    + bl2_ref[0:1, :], 0.0)
    z = _bn_rows(z, g2_ref[0:1, :], be2_ref[0:1, :])
    z = jnp.maximum(jnp.dot(z, wn3_ref[...],
                            preferred_element_type=jnp.float32)
                    + bl3_ref[0:1, :], 0.0)
    z = _bn_rows(z, g3_ref[0:1, :], be3_ref[0:1, :])
    z = jnp.dot(z, wn4_ref[...], preferred_element_type=jnp.float32) \
        + bl4_ref[0:1, :]
    zmax = jnp.max(z, axis=1, keepdims=True)
    zs = z - zmax
    lse = jnp.log(jnp.sum(jnp.exp(zs), axis=1, keepdims=True))
    out_ref[...] = zs - lse


def _head2(y1p, g1, be1, wn2, bl2, g2, be2, wn3, bl3, g3, be3, wn4, bl4):
    full = lambda s: pl.BlockSpec(s, lambda: tuple(0 for _ in s))
    args = (y1p, g1.reshape(1, -1), be1.reshape(1, -1), wn2,
            bl2.reshape(1, -1), g2.reshape(1, -1), be2.reshape(1, -1), wn3,
            bl3.reshape(1, -1), g3.reshape(1, -1), be3.reshape(1, -1), wn4,
            bl4.reshape(1, -1))
    return pl.pallas_call(
        _head2_body,
        in_specs=[full(a.shape) for a in args],
        out_specs=full((NB, 40)),
        out_shape=jax.ShapeDtypeStruct((NB, 40), jnp.float32),
        interpret=_INTERPRET,
    )(*args)


def kernel(pos, edge_index, batch, W1, b1, W2, b2, gdd, bdd, Wn1, bl1, g1,
           be1, Wn2, bl2, g2, be2, Wn3, bl3, g3, be3, Wn4, bl4):
    del edge_index, batch
    posf = jnp.pad(pos, ((0, 0), (0, 5)))
    pos8 = posf.reshape(NB, P, 8)
    pos8t = pos8.transpose(0, 2, 1)
    w1p = jnp.pad(W1, ((0, 5), (0, 0)))
    feat = _knn_msg(pos8, pos8t, w1p, b1.reshape(1, 64))
    g, fnorm = _pairmax(feat.reshape(NC, 2, 384), W2)
    fn8 = fnorm.reshape(NB, PC, 8)
    mx, mn, ps = _knn_h(fn8, fn8.transpose(0, 2, 1), g, b2.reshape(1, 128))
    y1p = _head1(mx, mn, ps, gdd.reshape(1, 128), bdd.reshape(1, 128), Wn1,
                 bl1.reshape(1, 1024)).reshape(NB, 1024)
    return _head2(y1p, g1, be1, Wn2, bl2, g2, be2, Wn3, bl3, g3, be3, Wn4,
                  bl4)
